# baseline (speedup 1.0000x reference)
import functools
import math

import jax
import jax.numpy as jnp
from jax import lax
from jax.experimental import pallas as pl
from jax.experimental.pallas import tpu as pltpu
from jax.experimental.pallas import tpu_sc as plsc

NHEAD = 4
SDIM = 10
_F32 = jnp.float32
_BF16 = jnp.bfloat16


def _split_bf16(a):
    hi = a.astype(_BF16)
    lo = (a - hi.astype(_F32)).astype(_BF16)
    return hi, lo


def _d(a, b, dims):
    return lax.dot_general(a, b, (dims, ((), ())),
                           preferred_element_type=_F32)


def _dot3(a, b, dims):
    ah, al = _split_bf16(a)
    if isinstance(b, tuple):
        bh, bl = b
    else:
        bh, bl = _split_bf16(b)
    return _d(al, bh, dims) + _d(ah, bl, dims) + _d(ah, bh, dims)


def _nt(a, b):
    return _dot3(a, b, ((1,), (1,)))


def _mm(a, b):
    return _dot3(a, b, ((1,), (0,)))


def _dotx(a, b, dims):
    return lax.dot_general(a, b, (dims, ((), ())),
                           precision=lax.Precision.HIGHEST,
                           preferred_element_type=_F32)


def _nt_x(a, b):
    return _dotx(a, b, ((1,), (1,)))


def _tn_x(a, b):
    return _dotx(a, b, ((0,), (0,)))


def _mm_x(a, b):
    return _dotx(a, b, ((1,), (0,)))


def _ln(w, x):
    m = jnp.mean(x, axis=-1, keepdims=True)
    v = jnp.mean((x - m) ** 2, axis=-1, keepdims=True)
    return (x - m) / jnp.sqrt(v + 1e-5) * w


def _softmax(x):
    m = jnp.max(x, axis=-1, keepdims=True)
    e = jnp.exp(x - m)
    return e / jnp.sum(e, axis=-1, keepdims=True)


def _mlp(w, x):
    return _mm(jax.nn.gelu(_mm(x, w['fc'])), w['pr'])


def _pair_rows(wpair, lo, hi):
    return (wpair[0][lo:hi], wpair[1][lo:hi])


def _attn_causal(w, x):
    t, c = x.shape
    hd = c // NHEAD
    q = _mm(x, w['wq'])
    k = _mm(x, w['wk'])
    v = _mm(x, w['wv'])
    ii = lax.broadcasted_iota(jnp.int32, (t, t), 0)
    jj = lax.broadcasted_iota(jnp.int32, (t, t), 1)
    mask = jj <= ii
    ys = []
    for h in range(NHEAD):
        sl = slice(h * hd, (h + 1) * hd)
        att = _nt(q[:, sl], k[:, sl]) / math.sqrt(float(hd))
        att = jnp.where(mask, att, -1e30)
        ys.append(_mm(_softmax(att), v[:, sl]))
    y = jnp.concatenate(ys, axis=1)
    return _mm(y, w['wo'])


def _block(w, x):
    x = x + _attn_causal(w, _ln(w['ln1'], x))
    x = x + _mlp(w, _ln(w['ln2'], x))
    return x


def _sblock(w, x, sx):
    xn = _ln(w['ln1'], x)
    sn = _ln(w['ln2'], sx)
    t, c = xn.shape
    hd = c // NHEAD
    q = _mm(sn, w['q'])
    k = _mm(xn, w['k'])
    v = _mm(xn, w['v'])
    ys = []
    for h in range(NHEAD):
        sl = slice(h * hd, (h + 1) * hd)
        att = _nt(q[:, sl], k[:, sl])
        ys.append(_mm(_softmax(att), v[:, sl]))
    y = jnp.concatenate(ys, axis=1)
    sx = sx + _mm(y, w['o'])
    sx = sx + _mlp(w, _ln(w['ln3'], sx))
    return sx


def _sel_matrix(g_col, kk):
    t = g_col.shape[0]
    ones_col = jnp.ones((t, 1), _F32)
    gj = _nt_x(ones_col, g_col)
    gi = jnp.broadcast_to(g_col, (t, t))
    ii = lax.broadcasted_iota(jnp.int32, (t, t), 0)
    jj = lax.broadcasted_iota(jnp.int32, (t, t), 1)
    beats = ((gj > gi) | ((gj == gi) & (jj < ii))).astype(_F32)
    rank = _mm_x(beats, ones_col)
    sel = rank < float(kk)
    lower = (jj < ii).astype(_F32)
    before = _mm_x(lower, sel.astype(_F32))
    s_iota = lax.broadcasted_iota(jnp.int32, (t, kk), 1).astype(_F32)
    q = jnp.where(sel & (jnp.broadcast_to(before, (t, kk)) == s_iota),
                  jnp.float32(1.0), jnp.float32(0.0))
    return q


def _gate_cols(x, scan):
    t = x.shape[0]
    xs = jnp.concatenate([x[1:], x[:1]], axis=0)
    d = xs - scan
    g = jnp.sum(d * d, axis=-1, keepdims=True)
    tcol = lax.broadcasted_iota(jnp.int32, (t, 1), 0)
    return jnp.where(tcol < t - 1, g, jnp.float32(0.0))


def _sgate_row(p, w3t, sgb):
    acc = sgb
    for d in range(SDIM):
        acc = acc + jnp.sum(p[:, d:d + 1] * w3t[d], axis=0, keepdims=True)
    return acc


def _row_to_col(r):
    ones11 = jnp.ones((1, 1), _F32)
    return _tn_x(r, ones11)


def _wsplit(w):
    hi = w.astype(_BF16)
    lo = (w - hi.astype(_F32)).astype(_BF16)
    return (hi, lo)


def _block_w(p):
    return {'ln1': p['ln1'].reshape(1, -1), 'ln2': p['ln2'].reshape(1, -1),
            'wq': _wsplit(p['wq']['w']), 'wk': _wsplit(p['wk']['w']),
            'wv': _wsplit(p['wv']['w']), 'wo': _wsplit(p['wo']['w']),
            'fc': _wsplit(p['fc']['w']), 'pr': _wsplit(p['pr']['w'])}


def _sblock_w(p):
    return {'ln1': p['ln1'].reshape(1, -1), 'ln2': p['ln2'].reshape(1, -1),
            'ln3': p['ln3'].reshape(1, -1),
            'q': _wsplit(p['q']['w']), 'k': _wsplit(p['k']['w']),
            'v': _wsplit(p['v']['w']), 'o': _wsplit(p['o']['w']),
            'fc': _wsplit(p['fc']['w']), 'pr': _wsplit(p['pr']['w'])}


_BLOCK_KEYS = ('ln1', 'ln2', 'wq', 'wk', 'wv', 'wo', 'fc', 'pr')
_SBLOCK_KEYS = ('ln1', 'ln2', 'ln3', 'q', 'k', 'v', 'o', 'fc', 'pr')


def _flat(vals):
    out = []
    for v in vals:
        if isinstance(v, tuple):
            out.extend(v)
        else:
            out.append(v)
    return out


def _wspec_for(vals):
    return [_wspec(v.shape) for v in _flat(vals)]


def _load(template, it):
    out = []
    for v in template:
        if isinstance(v, tuple):
            h = next(it)[...]
            l = next(it)[...]
            out.append((h, l))
        else:
            out.append(next(it)[...])
    return out


def _wspec(shape):
    nd = len(shape)
    return pl.BlockSpec(shape, lambda b: (0,) * nd)


def _bspec(shape3):
    return pl.BlockSpec((1,) + shape3[1:], lambda b: (b, 0, 0))


def _sc_embed(table, idx_flat):
    n = idx_flat.shape[0]
    d = table.shape[1]
    info = plsc.get_sparse_core_info()
    nc, ns = info.num_cores, info.num_subcores
    nw = nc * ns
    per_w = n // nw
    mesh = plsc.VectorSubcoreMesh(core_axis_name="c", subcore_axis_name="s")

    @functools.partial(
        pl.kernel, mesh=mesh,
        out_type=jax.ShapeDtypeStruct((nw, per_w, d), _F32),
        compiler_params=pltpu.CompilerParams(use_tc_tiling_on_sc=True),
        scratch_types=[
            pltpu.VMEM((per_w,), jnp.int32),
            pltpu.VMEM((per_w, d), _F32),
            pltpu.SemaphoreType.DMA,
        ],
    )
    def emb_k(table_hbm, idx_hbm, out_hbm, idx_v, rows_v, sem):
        wid = lax.axis_index("s") * nc + lax.axis_index("c")
        base = wid * per_w
        pltpu.sync_copy(idx_hbm.at[pl.ds(base, per_w)], idx_v)
        pltpu.async_copy(table_hbm.at[idx_v], rows_v, sem).wait()
        pltpu.sync_copy(rows_v, out_hbm.at[wid])

    return emb_k(table, idx_flat).reshape(n, d)


def _k_up(x, pos, bw, upw, upb, iseq, add_pos, c_used=None, par=None):
    bsz, t, c = x.shape
    if c_used is None:
        c_used = c
    idim = upw[0].shape[1]
    wvals = [bw[kk] for kk in _BLOCK_KEYS] + [upw, upb]
    if par is None:
        par = jnp.zeros((bsz, t, 1), _F32)

    def body(x_ref, pos_ref, par_ref, *rest):
        it = iter(rest[:-2])
        loaded = _load(wvals, it)
        scan_ref, pup_ref = rest[-2], rest[-1]
        w = dict(zip(_BLOCK_KEYS, loaded[:-2]))
        upw_v, upb_v = loaded[-2], loaded[-1]
        if c != c_used:
            xx = jnp.where(par_ref[0] > 0.5,
                           x_ref[0][:, c_used:2 * c_used],
                           x_ref[0][:, 0:c_used])
        else:
            xx = x_ref[0]
        if add_pos:
            xx = xx + pos_ref[...]
        scan = _block(w, xx)
        scan_ref[0] = scan
        g = _gate_cols(xx, scan)
        q = _sel_matrix(g, iseq)
        gathered = _tn_x(q, _softmax(scan))
        pup_ref[0] = _mm(gathered, upw_v) + upb_v

    in_specs = ([_bspec(x.shape), _wspec(pos.shape), _bspec(par.shape)]
                + _wspec_for(wvals))
    return pl.pallas_call(
        body,
        grid=(bsz,),
        in_specs=in_specs,
        out_specs=[_bspec((bsz, t, c_used)), _bspec((bsz, iseq, idim))],
        out_shape=[jax.ShapeDtypeStruct((bsz, t, c_used), _F32),
                   jax.ShapeDtypeStruct((bsz, iseq, idim), _F32)],
    )(x, pos, par, *_flat(wvals))


def _k_attn_flat(x, bw, blk):
    n, c = x.shape
    hd = c // NHEAD
    wvals = [bw['ln1'], bw['wq'], bw['wk'], bw['wv'], bw['wo']]

    def body(x_ref, *rest):
        loaded = _load(wvals, iter(rest[:-1]))
        out_ref = rest[-1]
        ln1, wq, wk, wv, wo = loaded
        x_v = x_ref[...]
        xn = _ln(ln1, x_v)
        q = _mm(xn, wq)
        k = _mm(xn, wk)
        v = _mm(xn, wv)
        ii = lax.broadcasted_iota(jnp.int32, (n, n), 0)
        jj = lax.broadcasted_iota(jnp.int32, (n, n), 1)
        mask = (ii // blk == jj // blk) & (jj <= ii)
        acc = x_v
        for h in range(NHEAD):
            sl = slice(h * hd, (h + 1) * hd)
            att = _nt(q[:, sl], k[:, sl]) / math.sqrt(float(hd))
            att = jnp.where(mask, att, -1e30)
            yh = _mm(_softmax(att), v[:, sl])
            acc = acc + _mm(yh, _pair_rows(wo, h * hd, (h + 1) * hd))
        out_ref[...] = acc

    return pl.pallas_call(
        body,
        grid=(1,),
        in_specs=[_wspec(x.shape)] + _wspec_for(wvals),
        out_specs=_wspec((n, c)),
        out_shape=jax.ShapeDtypeStruct((n, c), _F32),
    )(x, *_flat(wvals))


def _k_xattn_flat(xkv, sx, sbw, qblk, kvblk):
    nk, c = xkv.shape
    nq = sx.shape[0]
    hd = c // NHEAD
    wvals = [sbw['ln1'], sbw['ln2'], sbw['q'], sbw['k'], sbw['v'], sbw['o']]

    def body(xkv_ref, sx_ref, *rest):
        loaded = _load(wvals, iter(rest[:-1]))
        out_ref = rest[-1]
        ln1, ln2, wq, wk, wv, wo = loaded
        sx_v = sx_ref[...]
        xn = _ln(ln1, xkv_ref[...])
        sn = _ln(ln2, sx_v)
        q = _mm(sn, wq)
        k = _mm(xn, wk)
        v = _mm(xn, wv)
        ii = lax.broadcasted_iota(jnp.int32, (nq, nk), 0)
        jj = lax.broadcasted_iota(jnp.int32, (nq, nk), 1)
        mask = (ii // qblk) == (jj // kvblk)
        acc = sx_v
        for h in range(NHEAD):
            sl = slice(h * hd, (h + 1) * hd)
            att = _nt(q[:, sl], k[:, sl])
            att = jnp.where(mask, att, -1e30)
            yh = _mm(_softmax(att), v[:, sl])
            acc = acc + _mm(yh, _pair_rows(wo, h * hd, (h + 1) * hd))
        out_ref[...] = acc

    return pl.pallas_call(
        body,
        grid=(1,),
        in_specs=[_wspec(xkv.shape), _wspec(sx.shape)] + _wspec_for(wvals),
        out_specs=_wspec((nq, c)),
        out_shape=jax.ShapeDtypeStruct((nq, c), _F32),
    )(xkv, sx, *_flat(wvals))


def _k_mlp_tiled(x, lnw, fc, pr, ntiles):
    n, c = x.shape
    hdim = fc[0].shape[1]
    ht = hdim // ntiles

    def body(x_ref, lnw_ref, fch_ref, fcl_ref, prh_ref, prl_ref, out_ref):
        t = pl.program_id(0)
        xn = _ln(lnw_ref[...], x_ref[...])
        hcur = jax.nn.gelu(_mm(xn, (fch_ref[...], fcl_ref[...])))
        term = _mm(hcur, (prh_ref[...], prl_ref[...]))

        @pl.when(t == 0)
        def _():
            out_ref[...] = x_ref[...] + term

        @pl.when(t != 0)
        def _():
            out_ref[...] = out_ref[...] + term

    return pl.pallas_call(
        body,
        grid=(ntiles,),
        in_specs=[
            _wspec(x.shape), _wspec(lnw.shape),
            pl.BlockSpec((c, ht), lambda t: (0, t)),
            pl.BlockSpec((c, ht), lambda t: (0, t)),
            pl.BlockSpec((ht, c), lambda t: (t, 0)),
            pl.BlockSpec((ht, c), lambda t: (t, 0)),
        ],
        out_specs=_wspec((n, c)),
        out_shape=jax.ShapeDtypeStruct((n, c), _F32),
    )(x, lnw, fc[0], fc[1], pr[0], pr[1])


def _k_scatter(passed, w3t, sgb, posw, posb, oseq):
    bsz, iseq, idim = passed.shape

    def body(p_ref, w3t_ref, sgb_ref, posw_ref, posb_ref, out_ref):
        p = p_ref[0]
        sg_row = _sgate_row(p, w3t_ref[...], sgb_ref[...])
        sg_col = _row_to_col(sg_row)
        q = _sel_matrix(sg_col, iseq)
        out_ref[0] = _mm_x(q, p) + posw_ref[...] + posb_ref[...]

    return pl.pallas_call(
        body,
        grid=(bsz,),
        in_specs=[_bspec(passed.shape), _wspec(w3t.shape), _wspec(sgb.shape),
                  _wspec(posw.shape), _wspec(posb.shape)],
        out_specs=_bspec((bsz, oseq, idim)),
        out_shape=jax.ShapeDtypeStruct((bsz, oseq, idim), _F32),
    )(passed, w3t, sgb, posw, posb)


def _k_proj_block(pds, scan, dpw, dpb, dsw):
    bsz, t, idim = pds.shape
    od = scan.shape[2]
    wvals = [dpw, dpb] + [dsw[kk] for kk in _BLOCK_KEYS]

    def body(pds_ref, scan_ref, *rest):
        loaded = _load(wvals, iter(rest[:-1]))
        out_ref = rest[-1]
        dpw_v, dpb_v = loaded[0], loaded[1]
        dw = dict(zip(_BLOCK_KEYS, loaded[2:]))
        x = _mm(pds_ref[0], dpw_v) + dpb_v + scan_ref[0]
        out_ref[0] = _block(dw, x)

    return pl.pallas_call(
        body,
        grid=(bsz,),
        in_specs=[_bspec(pds.shape), _bspec(scan.shape)] + _wspec_for(wvals),
        out_specs=_bspec((bsz, t, od)),
        out_shape=jax.ShapeDtypeStruct((bsz, t, od), _F32),
    )(pds, scan, *_flat(wvals))


def _k_down_last(passed, scan, w3t, sgb, posw, posb, sbw, dpw, dpb, dsw):
    bsz, iseq, idim = passed.shape
    _, oseq, od = scan.shape
    wvals = ([w3t, sgb, posw, posb]
             + [sbw[kk] for kk in _SBLOCK_KEYS]
             + [dpw, dpb]
             + [dsw[kk] for kk in _BLOCK_KEYS])

    def body(p_ref, scan_ref, *rest):
        loaded = _load(wvals, iter(rest[:-1]))
        out_ref = rest[-1]
        w3t_v, sgb_v, posw_v, posb_v = loaded[:4]
        sw = dict(zip(_SBLOCK_KEYS, loaded[4:4 + len(_SBLOCK_KEYS)]))
        dpw_v, dpb_v = loaded[4 + len(_SBLOCK_KEYS):6 + len(_SBLOCK_KEYS)]
        dw = dict(zip(_BLOCK_KEYS, loaded[6 + len(_SBLOCK_KEYS):]))
        p = p_ref[0]
        sg_row = _sgate_row(p, w3t_v, sgb_v)
        sg_col = _row_to_col(sg_row)
        q = _sel_matrix(sg_col, iseq)
        scattered = _mm_x(q, p) + posw_v + posb_v
        pds = _sblock(sw, p, scattered)
        x = _mm(pds, dpw_v) + dpb_v + scan_ref[0]
        xn = _ln(dw['ln1'], x)
        hd = od // NHEAD
        k = _mm(xn, dw['wk'])
        v = _mm(xn, dw['wv'])
        q_last = _mm(xn[oseq - 1:oseq], dw['wq'])
        ys = []
        for h in range(NHEAD):
            sl = slice(h * hd, (h + 1) * hd)
            att = _nt(q_last[:, sl], k[:, sl]) / math.sqrt(float(hd))
            ys.append(_mm(_softmax(att), v[:, sl]))
        y = jnp.concatenate(ys, axis=1)
        xl = x[oseq - 1:oseq] + _mm(y, dw['wo'])
        out_ref[0] = xl + _mlp(dw, _ln(dw['ln2'], xl))

    return pl.pallas_call(
        body,
        grid=(bsz,),
        in_specs=[_bspec(passed.shape), _bspec(scan.shape)]
        + _wspec_for(wvals),
        out_specs=_bspec((bsz, 1, od)),
        out_shape=jax.ShapeDtypeStruct((bsz, 1, od), _F32),
    )(passed, scan, *_flat(wvals))


def _k_head(x, hw, hb):
    bsz, c = x.shape
    vocab = hw[0].shape[1]

    def body(x_ref, hwh_ref, hwl_ref, hb_ref, out_ref):
        out_ref[...] = _mm(x_ref[...], (hwh_ref[...], hwl_ref[...])) \
            + hb_ref[...]

    return pl.pallas_call(
        body,
        grid=(1,),
        in_specs=[_wspec(x.shape), _wspec(hw[0].shape), _wspec(hw[1].shape),
                  _wspec(hb.shape)],
        out_specs=_wspec((bsz, vocab)),
        out_shape=jax.ShapeDtypeStruct((bsz, vocab), _F32),
    )(x, hw[0], hw[1], hb)


def kernel(params, idx):
    bsz, t = idx.shape
    l1, l2 = params['L1'], params['L2']
    dim1 = params['tok_emb'].shape[1]
    k1 = l2['pos']['w'].shape[0]
    k2 = l2['s_gate']['w'].shape[0] // SDIM
    dim3 = l2['up_proj']['w'].shape[1]
    dim2 = l1['up_proj']['w'].shape[1]

    vhalf = params['tok_emb'].shape[0] // 2
    table2 = params['tok_emb'].reshape(vhalf, 2 * dim1)
    idxf = idx.reshape(-1).astype(jnp.int32)
    emb = _sc_embed(table2, idxf // 2)
    emb = emb.reshape(bsz, t, 2 * dim1)
    par = jnp.remainder(idxf, 2).astype(_F32).reshape(bsz, t, 1)

    scan1, p_up1 = _k_up(
        emb, params['pos_emb'], _block_w(l1['up_scan']),
        _wsplit(l1['up_proj']['w']), l1['up_proj']['b'].reshape(1, -1),
        iseq=k1, add_pos=True, c_used=dim1, par=par)

    dummy_pos = jnp.zeros((1, 1), _F32)
    scan2, p_up2 = _k_up(
        p_up1, dummy_pos, _block_w(l2['up_scan']),
        _wsplit(l2['up_proj']['w']), l2['up_proj']['b'].reshape(1, -1),
        iseq=k2, add_pos=False)

    ib = _block_w(params['innerb'])
    xf = p_up2.reshape(bsz * k2, dim3)
    xf = _k_attn_flat(xf, ib, blk=k2)
    xf = _k_mlp_tiled(xf, ib['ln2'], ib['fc'], ib['pr'], ntiles=4)
    passed2 = xf.reshape(bsz, k2, dim3)

    w3t2 = l2['s_gate']['w'].reshape(k2, SDIM, k1).transpose(1, 0, 2)
    scattered2 = _k_scatter(
        passed2, w3t2, l2['s_gate']['b'].reshape(1, -1),
        l2['pos']['w'], l2['pos']['b'].reshape(1, -1), oseq=k1)
    sb2 = _sblock_w(l2['down_scatter'])
    sx1 = _k_xattn_flat(xf, scattered2.reshape(bsz * k1, dim3), sb2,
                        qblk=k1, kvblk=k2)
    pds2 = _k_mlp_tiled(sx1, sb2['ln3'], sb2['fc'], sb2['pr'], ntiles=4)
    passed1 = _k_proj_block(
        pds2.reshape(bsz, k1, dim3), scan2,
        _wsplit(l2['down_proj']['w']), l2['down_proj']['b'].reshape(1, -1),
        _block_w(l2['down_scan']))

    w3t1 = l1['s_gate']['w'].reshape(k1, SDIM, t).transpose(1, 0, 2)
    out_last = _k_down_last(
        passed1, scan1, w3t1, l1['s_gate']['b'].reshape(1, -1),
        l1['pos']['w'], l1['pos']['b'].reshape(1, -1),
        _sblock_w(l1['down_scatter']),
        _wsplit(l1['down_proj']['w']), l1['down_proj']['b'].reshape(1, -1),
        _block_w(l1['down_scan']))

    logits = _k_head(out_last.reshape(bsz, dim1),
                     _wsplit(params['head']['w']),
                     params['head']['b'].reshape(1, -1))
    return logits.reshape(bsz, 1, logits.shape[1])


# speedup vs baseline: 1.0041x; 1.0041x over previous
import functools
import math

import jax
import jax.numpy as jnp
from jax import lax
from jax.experimental import pallas as pl
from jax.experimental.pallas import tpu as pltpu
from jax.experimental.pallas import tpu_sc as plsc

NHEAD = 4
SDIM = 10
_F32 = jnp.float32
_BF16 = jnp.bfloat16


def _split_bf16(a):
    hi = a.astype(_BF16)
    lo = (a - hi.astype(_F32)).astype(_BF16)
    return hi, lo


def _d(a, b, dims):
    return lax.dot_general(a, b, (dims, ((), ())),
                           preferred_element_type=_F32)


def _dot3(a, b, dims):
    ah, al = _split_bf16(a)
    if isinstance(b, tuple):
        bh, bl = b
    else:
        bh, bl = _split_bf16(b)
    return _d(al, bh, dims) + _d(ah, bl, dims) + _d(ah, bh, dims)


def _nt(a, b):
    return _dot3(a, b, ((1,), (1,)))


def _mm(a, b):
    return _dot3(a, b, ((1,), (0,)))


def _dotx(a, b, dims):
    return lax.dot_general(a, b, (dims, ((), ())),
                           precision=lax.Precision.HIGHEST,
                           preferred_element_type=_F32)


def _nt_x(a, b):
    return _dotx(a, b, ((1,), (1,)))


def _tn_x(a, b):
    return _dotx(a, b, ((0,), (0,)))


def _mm_x(a, b):
    return _dotx(a, b, ((1,), (0,)))


def _ln(w, x):
    m = jnp.mean(x, axis=-1, keepdims=True)
    v = jnp.mean((x - m) ** 2, axis=-1, keepdims=True)
    return (x - m) / jnp.sqrt(v + 1e-5) * w


def _softmax(x):
    m = jnp.max(x, axis=-1, keepdims=True)
    e = jnp.exp(x - m)
    return e / jnp.sum(e, axis=-1, keepdims=True)


def _mlp(w, x):
    return _mm(jax.nn.gelu(_mm(x, w['fc'])), w['pr'])


def _pair_rows(wpair, lo, hi):
    return (wpair[0][lo:hi], wpair[1][lo:hi])


def _attn_causal(w, x):
    t, c = x.shape
    hd = c // NHEAD
    q = _mm(x, w['wq'])
    k = _mm(x, w['wk'])
    v = _mm(x, w['wv'])
    ii = lax.broadcasted_iota(jnp.int32, (t, t), 0)
    jj = lax.broadcasted_iota(jnp.int32, (t, t), 1)
    mask = jj <= ii
    ys = []
    for h in range(NHEAD):
        sl = slice(h * hd, (h + 1) * hd)
        att = _nt(q[:, sl], k[:, sl]) / math.sqrt(float(hd))
        att = jnp.where(mask, att, -1e30)
        ys.append(_mm(_softmax(att), v[:, sl]))
    y = jnp.concatenate(ys, axis=1)
    return _mm(y, w['wo'])


def _block(w, x):
    x = x + _attn_causal(w, _ln(w['ln1'], x))
    x = x + _mlp(w, _ln(w['ln2'], x))
    return x


def _sblock(w, x, sx):
    xn = _ln(w['ln1'], x)
    sn = _ln(w['ln2'], sx)
    t, c = xn.shape
    hd = c // NHEAD
    q = _mm(sn, w['q'])
    k = _mm(xn, w['k'])
    v = _mm(xn, w['v'])
    ys = []
    for h in range(NHEAD):
        sl = slice(h * hd, (h + 1) * hd)
        att = _nt(q[:, sl], k[:, sl])
        ys.append(_mm(_softmax(att), v[:, sl]))
    y = jnp.concatenate(ys, axis=1)
    sx = sx + _mm(y, w['o'])
    sx = sx + _mlp(w, _ln(w['ln3'], sx))
    return sx


def _sel_matrix(g_col, kk):
    t = g_col.shape[0]
    ones_col = jnp.ones((t, 1), _F32)
    gj = _nt_x(ones_col, g_col)
    gi = jnp.broadcast_to(g_col, (t, t))
    ii = lax.broadcasted_iota(jnp.int32, (t, t), 0)
    jj = lax.broadcasted_iota(jnp.int32, (t, t), 1)
    beats = ((gj > gi) | ((gj == gi) & (jj < ii))).astype(_F32)
    rank = _mm_x(beats, ones_col)
    sel = rank < float(kk)
    lower = (jj < ii).astype(_F32)
    before = _mm_x(lower, sel.astype(_F32))
    s_iota = lax.broadcasted_iota(jnp.int32, (t, kk), 1).astype(_F32)
    q = jnp.where(sel & (jnp.broadcast_to(before, (t, kk)) == s_iota),
                  jnp.float32(1.0), jnp.float32(0.0))
    return q


def _gate_cols(x, scan):
    t = x.shape[0]
    xs = jnp.concatenate([x[1:], x[:1]], axis=0)
    d = xs - scan
    g = jnp.sum(d * d, axis=-1, keepdims=True)
    tcol = lax.broadcasted_iota(jnp.int32, (t, 1), 0)
    return jnp.where(tcol < t - 1, g, jnp.float32(0.0))


def _sgate_row(p, w3t, sgb):
    acc = sgb
    for d in range(SDIM):
        acc = acc + jnp.sum(p[:, d:d + 1] * w3t[d], axis=0, keepdims=True)
    return acc


def _row_to_col(r):
    ones11 = jnp.ones((1, 1), _F32)
    return _tn_x(r, ones11)


def _wsplit(w):
    hi = w.astype(_BF16)
    lo = (w - hi.astype(_F32)).astype(_BF16)
    return (hi, lo)


def _block_w(p):
    return {'ln1': p['ln1'].reshape(1, -1), 'ln2': p['ln2'].reshape(1, -1),
            'wq': _wsplit(p['wq']['w']), 'wk': _wsplit(p['wk']['w']),
            'wv': _wsplit(p['wv']['w']), 'wo': _wsplit(p['wo']['w']),
            'fc': _wsplit(p['fc']['w']), 'pr': _wsplit(p['pr']['w'])}


def _sblock_w(p):
    return {'ln1': p['ln1'].reshape(1, -1), 'ln2': p['ln2'].reshape(1, -1),
            'ln3': p['ln3'].reshape(1, -1),
            'q': _wsplit(p['q']['w']), 'k': _wsplit(p['k']['w']),
            'v': _wsplit(p['v']['w']), 'o': _wsplit(p['o']['w']),
            'fc': _wsplit(p['fc']['w']), 'pr': _wsplit(p['pr']['w'])}


_BLOCK_KEYS = ('ln1', 'ln2', 'wq', 'wk', 'wv', 'wo', 'fc', 'pr')
_SBLOCK_KEYS = ('ln1', 'ln2', 'ln3', 'q', 'k', 'v', 'o', 'fc', 'pr')


def _flat(vals):
    out = []
    for v in vals:
        if isinstance(v, tuple):
            out.extend(v)
        else:
            out.append(v)
    return out


def _wspec_for(vals):
    return [_wspec(v.shape) for v in _flat(vals)]


def _load(template, it):
    out = []
    for v in template:
        if isinstance(v, tuple):
            h = next(it)[...]
            l = next(it)[...]
            out.append((h, l))
        else:
            out.append(next(it)[...])
    return out


def _wspec(shape):
    nd = len(shape)
    return pl.BlockSpec(shape, lambda b: (0,) * nd)


def _bspec(shape3):
    return pl.BlockSpec((1,) + shape3[1:], lambda b: (b, 0, 0))


def _sc_embed(table, idx_flat):
    n = idx_flat.shape[0]
    d = table.shape[1]
    info = plsc.get_sparse_core_info()
    nc, ns = info.num_cores, info.num_subcores
    nw = nc * ns
    per_w = n // nw
    mesh = plsc.VectorSubcoreMesh(core_axis_name="c", subcore_axis_name="s")

    @functools.partial(
        pl.kernel, mesh=mesh,
        out_type=jax.ShapeDtypeStruct((n, d), _F32),
        scratch_types=[
            pltpu.VMEM((per_w,), jnp.int32),
            pltpu.VMEM((per_w, d), _F32),
            pltpu.SemaphoreType.DMA,
        ],
    )
    def emb_k(table_hbm, idx_hbm, out_hbm, idx_v, rows_v, sem):
        wid = lax.axis_index("s") * nc + lax.axis_index("c")
        base = wid * per_w
        pltpu.sync_copy(idx_hbm.at[pl.ds(base, per_w)], idx_v)
        pltpu.async_copy(table_hbm.at[idx_v], rows_v, sem).wait()
        pltpu.sync_copy(rows_v, out_hbm.at[pl.ds(base, per_w)])

    return emb_k(table, idx_flat)


def _k_up(x, pos, bw, upw, upb, iseq, add_pos, c_used=None, par=None):
    bsz, t, c = x.shape
    if c_used is None:
        c_used = c
    idim = upw[0].shape[1]
    wvals = [bw[kk] for kk in _BLOCK_KEYS] + [upw, upb]
    if par is None:
        par = jnp.zeros((bsz, t, 1), _F32)

    def body(x_ref, pos_ref, par_ref, *rest):
        it = iter(rest[:-2])
        loaded = _load(wvals, it)
        scan_ref, pup_ref = rest[-2], rest[-1]
        w = dict(zip(_BLOCK_KEYS, loaded[:-2]))
        upw_v, upb_v = loaded[-2], loaded[-1]
        if c != c_used:
            xx = jnp.where(par_ref[0] > 0.5,
                           x_ref[0][:, c_used:2 * c_used],
                           x_ref[0][:, 0:c_used])
        else:
            xx = x_ref[0]
        if add_pos:
            xx = xx + pos_ref[...]
        scan = _block(w, xx)
        scan_ref[0] = scan
        g = _gate_cols(xx, scan)
        q = _sel_matrix(g, iseq)
        gathered = _tn_x(q, _softmax(scan))
        pup_ref[0] = _mm(gathered, upw_v) + upb_v

    in_specs = ([_bspec(x.shape), _wspec(pos.shape), _bspec(par.shape)]
                + _wspec_for(wvals))
    return pl.pallas_call(
        body,
        grid=(bsz,),
        in_specs=in_specs,
        out_specs=[_bspec((bsz, t, c_used)), _bspec((bsz, iseq, idim))],
        out_shape=[jax.ShapeDtypeStruct((bsz, t, c_used), _F32),
                   jax.ShapeDtypeStruct((bsz, iseq, idim), _F32)],
    )(x, pos, par, *_flat(wvals))


def _k_attn_flat(x, bw, blk):
    n, c = x.shape
    hd = c // NHEAD
    wvals = [bw['ln1'], bw['wq'], bw['wk'], bw['wv'], bw['wo']]

    def body(x_ref, *rest):
        loaded = _load(wvals, iter(rest[:-1]))
        out_ref = rest[-1]
        ln1, wq, wk, wv, wo = loaded
        x_v = x_ref[...]
        xn = _ln(ln1, x_v)
        q = _mm(xn, wq)
        k = _mm(xn, wk)
        v = _mm(xn, wv)
        ii = lax.broadcasted_iota(jnp.int32, (n, n), 0)
        jj = lax.broadcasted_iota(jnp.int32, (n, n), 1)
        mask = (ii // blk == jj // blk) & (jj <= ii)
        acc = x_v
        for h in range(NHEAD):
            sl = slice(h * hd, (h + 1) * hd)
            att = _nt(q[:, sl], k[:, sl]) / math.sqrt(float(hd))
            att = jnp.where(mask, att, -1e30)
            yh = _mm(_softmax(att), v[:, sl])
            acc = acc + _mm(yh, _pair_rows(wo, h * hd, (h + 1) * hd))
        out_ref[...] = acc

    return pl.pallas_call(
        body,
        grid=(1,),
        in_specs=[_wspec(x.shape)] + _wspec_for(wvals),
        out_specs=_wspec((n, c)),
        out_shape=jax.ShapeDtypeStruct((n, c), _F32),
    )(x, *_flat(wvals))


def _k_xattn_flat(xkv, sx, sbw, qblk, kvblk):
    nk, c = xkv.shape
    nq = sx.shape[0]
    hd = c // NHEAD
    wvals = [sbw['ln1'], sbw['ln2'], sbw['q'], sbw['k'], sbw['v'], sbw['o']]

    def body(xkv_ref, sx_ref, *rest):
        loaded = _load(wvals, iter(rest[:-1]))
        out_ref = rest[-1]
        ln1, ln2, wq, wk, wv, wo = loaded
        sx_v = sx_ref[...]
        xn = _ln(ln1, xkv_ref[...])
        sn = _ln(ln2, sx_v)
        q = _mm(sn, wq)
        k = _mm(xn, wk)
        v = _mm(xn, wv)
        ii = lax.broadcasted_iota(jnp.int32, (nq, nk), 0)
        jj = lax.broadcasted_iota(jnp.int32, (nq, nk), 1)
        mask = (ii // qblk) == (jj // kvblk)
        acc = sx_v
        for h in range(NHEAD):
            sl = slice(h * hd, (h + 1) * hd)
            att = _nt(q[:, sl], k[:, sl])
            att = jnp.where(mask, att, -1e30)
            yh = _mm(_softmax(att), v[:, sl])
            acc = acc + _mm(yh, _pair_rows(wo, h * hd, (h + 1) * hd))
        out_ref[...] = acc

    return pl.pallas_call(
        body,
        grid=(1,),
        in_specs=[_wspec(xkv.shape), _wspec(sx.shape)] + _wspec_for(wvals),
        out_specs=_wspec((nq, c)),
        out_shape=jax.ShapeDtypeStruct((nq, c), _F32),
    )(xkv, sx, *_flat(wvals))


def _k_mlp_tiled(x, lnw, fc, pr, ntiles):
    n, c = x.shape
    hdim = fc[0].shape[1]
    ht = hdim // ntiles

    def body(x_ref, lnw_ref, fch_ref, fcl_ref, prh_ref, prl_ref, out_ref):
        t = pl.program_id(0)
        xn = _ln(lnw_ref[...], x_ref[...])
        hcur = jax.nn.gelu(_mm(xn, (fch_ref[...], fcl_ref[...])))
        term = _mm(hcur, (prh_ref[...], prl_ref[...]))

        @pl.when(t == 0)
        def _():
            out_ref[...] = x_ref[...] + term

        @pl.when(t != 0)
        def _():
            out_ref[...] = out_ref[...] + term

    return pl.pallas_call(
        body,
        grid=(ntiles,),
        in_specs=[
            _wspec(x.shape), _wspec(lnw.shape),
            pl.BlockSpec((c, ht), lambda t: (0, t)),
            pl.BlockSpec((c, ht), lambda t: (0, t)),
            pl.BlockSpec((ht, c), lambda t: (t, 0)),
            pl.BlockSpec((ht, c), lambda t: (t, 0)),
        ],
        out_specs=_wspec((n, c)),
        out_shape=jax.ShapeDtypeStruct((n, c), _F32),
    )(x, lnw, fc[0], fc[1], pr[0], pr[1])


def _k_scatter(passed, w3t, sgb, posw, posb, oseq):
    bsz, iseq, idim = passed.shape

    def body(p_ref, w3t_ref, sgb_ref, posw_ref, posb_ref, out_ref):
        p = p_ref[0]
        sg_row = _sgate_row(p, w3t_ref[...], sgb_ref[...])
        sg_col = _row_to_col(sg_row)
        q = _sel_matrix(sg_col, iseq)
        out_ref[0] = _mm_x(q, p) + posw_ref[...] + posb_ref[...]

    return pl.pallas_call(
        body,
        grid=(bsz,),
        in_specs=[_bspec(passed.shape), _wspec(w3t.shape), _wspec(sgb.shape),
                  _wspec(posw.shape), _wspec(posb.shape)],
        out_specs=_bspec((bsz, oseq, idim)),
        out_shape=jax.ShapeDtypeStruct((bsz, oseq, idim), _F32),
    )(passed, w3t, sgb, posw, posb)


def _k_proj_block(pds, scan, dpw, dpb, dsw):
    bsz, t, idim = pds.shape
    od = scan.shape[2]
    wvals = [dpw, dpb] + [dsw[kk] for kk in _BLOCK_KEYS]

    def body(pds_ref, scan_ref, *rest):
        loaded = _load(wvals, iter(rest[:-1]))
        out_ref = rest[-1]
        dpw_v, dpb_v = loaded[0], loaded[1]
        dw = dict(zip(_BLOCK_KEYS, loaded[2:]))
        x = _mm(pds_ref[0], dpw_v) + dpb_v + scan_ref[0]
        out_ref[0] = _block(dw, x)

    return pl.pallas_call(
        body,
        grid=(bsz,),
        in_specs=[_bspec(pds.shape), _bspec(scan.shape)] + _wspec_for(wvals),
        out_specs=_bspec((bsz, t, od)),
        out_shape=jax.ShapeDtypeStruct((bsz, t, od), _F32),
    )(pds, scan, *_flat(wvals))


def _k_down_last(passed, scan, w3t, sgb, posw, posb, sbw, dpw, dpb, dsw):
    bsz, iseq, idim = passed.shape
    _, oseq, od = scan.shape
    wvals = ([w3t, sgb, posw, posb]
             + [sbw[kk] for kk in _SBLOCK_KEYS]
             + [dpw, dpb]
             + [dsw[kk] for kk in _BLOCK_KEYS])

    def body(p_ref, scan_ref, *rest):
        loaded = _load(wvals, iter(rest[:-1]))
        out_ref = rest[-1]
        w3t_v, sgb_v, posw_v, posb_v = loaded[:4]
        sw = dict(zip(_SBLOCK_KEYS, loaded[4:4 + len(_SBLOCK_KEYS)]))
        dpw_v, dpb_v = loaded[4 + len(_SBLOCK_KEYS):6 + len(_SBLOCK_KEYS)]
        dw = dict(zip(_BLOCK_KEYS, loaded[6 + len(_SBLOCK_KEYS):]))
        p = p_ref[0]
        sg_row = _sgate_row(p, w3t_v, sgb_v)
        sg_col = _row_to_col(sg_row)
        q = _sel_matrix(sg_col, iseq)
        scattered = _mm_x(q, p) + posw_v + posb_v
        pds = _sblock(sw, p, scattered)
        x = _mm(pds, dpw_v) + dpb_v + scan_ref[0]
        xn = _ln(dw['ln1'], x)
        hd = od // NHEAD
        k = _mm(xn, dw['wk'])
        v = _mm(xn, dw['wv'])
        q_last = _mm(xn[oseq - 1:oseq], dw['wq'])
        ys = []
        for h in range(NHEAD):
            sl = slice(h * hd, (h + 1) * hd)
            att = _nt(q_last[:, sl], k[:, sl]) / math.sqrt(float(hd))
            ys.append(_mm(_softmax(att), v[:, sl]))
        y = jnp.concatenate(ys, axis=1)
        xl = x[oseq - 1:oseq] + _mm(y, dw['wo'])
        out_ref[0] = xl + _mlp(dw, _ln(dw['ln2'], xl))

    return pl.pallas_call(
        body,
        grid=(bsz,),
        in_specs=[_bspec(passed.shape), _bspec(scan.shape)]
        + _wspec_for(wvals),
        out_specs=_bspec((bsz, 1, od)),
        out_shape=jax.ShapeDtypeStruct((bsz, 1, od), _F32),
    )(passed, scan, *_flat(wvals))


def _k_head(x, hw, hb):
    bsz, c = x.shape
    vocab = hw[0].shape[1]

    def body(x_ref, hwh_ref, hwl_ref, hb_ref, out_ref):
        out_ref[...] = _mm(x_ref[...], (hwh_ref[...], hwl_ref[...])) \
            + hb_ref[...]

    return pl.pallas_call(
        body,
        grid=(1,),
        in_specs=[_wspec(x.shape), _wspec(hw[0].shape), _wspec(hw[1].shape),
                  _wspec(hb.shape)],
        out_specs=_wspec((bsz, vocab)),
        out_shape=jax.ShapeDtypeStruct((bsz, vocab), _F32),
    )(x, hw[0], hw[1], hb)


def kernel(params, idx):
    bsz, t = idx.shape
    l1, l2 = params['L1'], params['L2']
    dim1 = params['tok_emb'].shape[1]
    k1 = l2['pos']['w'].shape[0]
    k2 = l2['s_gate']['w'].shape[0] // SDIM
    dim3 = l2['up_proj']['w'].shape[1]
    dim2 = l1['up_proj']['w'].shape[1]

    emb_w = max(128, dim1)
    table = jnp.pad(params['tok_emb'], ((0, 0), (0, emb_w - dim1)))
    emb = _sc_embed(table, idx.reshape(-1).astype(jnp.int32))
    emb = emb.reshape(bsz, t, emb_w)

    scan1, p_up1 = _k_up(
        emb, params['pos_emb'], _block_w(l1['up_scan']),
        _wsplit(l1['up_proj']['w']), l1['up_proj']['b'].reshape(1, -1),
        iseq=k1, add_pos=True, c_used=dim1)

    dummy_pos = jnp.zeros((1, 1), _F32)
    scan2, p_up2 = _k_up(
        p_up1, dummy_pos, _block_w(l2['up_scan']),
        _wsplit(l2['up_proj']['w']), l2['up_proj']['b'].reshape(1, -1),
        iseq=k2, add_pos=False)

    ib = _block_w(params['innerb'])
    xf = p_up2.reshape(bsz * k2, dim3)
    xf = _k_attn_flat(xf, ib, blk=k2)
    xf = _k_mlp_tiled(xf, ib['ln2'], ib['fc'], ib['pr'], ntiles=4)
    passed2 = xf.reshape(bsz, k2, dim3)

    w3t2 = l2['s_gate']['w'].reshape(k2, SDIM, k1).transpose(1, 0, 2)
    scattered2 = _k_scatter(
        passed2, w3t2, l2['s_gate']['b'].reshape(1, -1),
        l2['pos']['w'], l2['pos']['b'].reshape(1, -1), oseq=k1)
    sb2 = _sblock_w(l2['down_scatter'])
    sx1 = _k_xattn_flat(xf, scattered2.reshape(bsz * k1, dim3), sb2,
                        qblk=k1, kvblk=k2)
    pds2 = _k_mlp_tiled(sx1, sb2['ln3'], sb2['fc'], sb2['pr'], ntiles=4)
    passed1 = _k_proj_block(
        pds2.reshape(bsz, k1, dim3), scan2,
        _wsplit(l2['down_proj']['w']), l2['down_proj']['b'].reshape(1, -1),
        _block_w(l2['down_scan']))

    w3t1 = l1['s_gate']['w'].reshape(k1, SDIM, t).transpose(1, 0, 2)
    out_last = _k_down_last(
        passed1, scan1, w3t1, l1['s_gate']['b'].reshape(1, -1),
        l1['pos']['w'], l1['pos']['b'].reshape(1, -1),
        _sblock_w(l1['down_scatter']),
        _wsplit(l1['down_proj']['w']), l1['down_proj']['b'].reshape(1, -1),
        _block_w(l1['down_scan']))

    logits = _k_head(out_last.reshape(bsz, dim1),
                     _wsplit(params['head']['w']),
                     params['head']['b'].reshape(1, -1))
    return logits.reshape(bsz, 1, logits.shape[1])


# speedup vs baseline: 1.0542x; 1.0499x over previous
import functools
import math

import jax
import jax.numpy as jnp
from jax import lax
from jax.experimental import pallas as pl
from jax.experimental.pallas import tpu as pltpu
from jax.experimental.pallas import tpu_sc as plsc

NHEAD = 4
SDIM = 10
_F32 = jnp.float32
_BF16 = jnp.bfloat16


def _split_bf16(a):
    hi = a.astype(_BF16)
    lo = (a - hi.astype(_F32)).astype(_BF16)
    return hi, lo


def _d(a, b, dims):
    return lax.dot_general(a, b, (dims, ((), ())),
                           preferred_element_type=_F32)


def _dot3(a, b, dims):
    ah, al = _split_bf16(a)
    if isinstance(b, tuple):
        bh, bl = b
    else:
        bh, bl = _split_bf16(b)
    return _d(al, bh, dims) + _d(ah, bl, dims) + _d(ah, bh, dims)


def _nt(a, b):
    return _dot3(a, b, ((1,), (1,)))


def _mm(a, b):
    return _dot3(a, b, ((1,), (0,)))


def _dotx(a, b, dims):
    return lax.dot_general(a, b, (dims, ((), ())),
                           precision=lax.Precision.HIGHEST,
                           preferred_element_type=_F32)


def _nt_x(a, b):
    return _dotx(a, b, ((1,), (1,)))


def _tn_x(a, b):
    return _dotx(a, b, ((0,), (0,)))


def _mm_x(a, b):
    return _dotx(a, b, ((1,), (0,)))


def _ln(w, x):
    m = jnp.mean(x, axis=-1, keepdims=True)
    v = jnp.mean((x - m) ** 2, axis=-1, keepdims=True)
    return (x - m) / jnp.sqrt(v + 1e-5) * w


def _softmax(x):
    m = jnp.max(x, axis=-1, keepdims=True)
    e = jnp.exp(x - m)
    return e / jnp.sum(e, axis=-1, keepdims=True)


def _mlp(w, x):
    return _mm(jax.nn.gelu(_mm(x, w['fc'])), w['pr'])


def _pair_rows(wpair, lo, hi):
    return (wpair[0][lo:hi], wpair[1][lo:hi])


def _attn_causal(w, x):
    t, c = x.shape
    hd = c // NHEAD
    q = _mm(x, w['wq'])
    k = _mm(x, w['wk'])
    v = _mm(x, w['wv'])
    ii = lax.broadcasted_iota(jnp.int32, (t, t), 0)
    jj = lax.broadcasted_iota(jnp.int32, (t, t), 1)
    mask = jj <= ii
    ys = []
    for h in range(NHEAD):
        sl = slice(h * hd, (h + 1) * hd)
        att = _nt(q[:, sl], k[:, sl]) / math.sqrt(float(hd))
        att = jnp.where(mask, att, -1e30)
        ys.append(_mm(_softmax(att), v[:, sl]))
    y = jnp.concatenate(ys, axis=1)
    return _mm(y, w['wo'])


def _block(w, x):
    x = x + _attn_causal(w, _ln(w['ln1'], x))
    x = x + _mlp(w, _ln(w['ln2'], x))
    return x


def _sblock(w, x, sx):
    xn = _ln(w['ln1'], x)
    sn = _ln(w['ln2'], sx)
    t, c = xn.shape
    hd = c // NHEAD
    q = _mm(sn, w['q'])
    k = _mm(xn, w['k'])
    v = _mm(xn, w['v'])
    ys = []
    for h in range(NHEAD):
        sl = slice(h * hd, (h + 1) * hd)
        att = _nt(q[:, sl], k[:, sl])
        ys.append(_mm(_softmax(att), v[:, sl]))
    y = jnp.concatenate(ys, axis=1)
    sx = sx + _mm(y, w['o'])
    sx = sx + _mlp(w, _ln(w['ln3'], sx))
    return sx


def _sel_matrix(g_col, kk):
    t = g_col.shape[0]
    ones_col = jnp.ones((t, 1), _F32)
    gj = _nt_x(ones_col, g_col)
    gi = jnp.broadcast_to(g_col, (t, t))
    ii = lax.broadcasted_iota(jnp.int32, (t, t), 0)
    jj = lax.broadcasted_iota(jnp.int32, (t, t), 1)
    beats = ((gj > gi) | ((gj == gi) & (jj < ii))).astype(_F32)
    rank = _mm_x(beats, ones_col)
    sel = rank < float(kk)
    lower = (jj < ii).astype(_F32)
    before = _mm_x(lower, sel.astype(_F32))
    s_iota = lax.broadcasted_iota(jnp.int32, (t, kk), 1).astype(_F32)
    q = jnp.where(sel & (jnp.broadcast_to(before, (t, kk)) == s_iota),
                  jnp.float32(1.0), jnp.float32(0.0))
    return q


def _gate_cols(x, scan):
    t = x.shape[0]
    xs = jnp.concatenate([x[1:], x[:1]], axis=0)
    d = xs - scan
    g = jnp.sum(d * d, axis=-1, keepdims=True)
    tcol = lax.broadcasted_iota(jnp.int32, (t, 1), 0)
    return jnp.where(tcol < t - 1, g, jnp.float32(0.0))


def _sgate_row(p, w3t, sgb):
    acc = sgb
    for d in range(SDIM):
        acc = acc + jnp.sum(p[:, d:d + 1] * w3t[d], axis=0, keepdims=True)
    return acc


def _row_to_col(r):
    ones11 = jnp.ones((1, 1), _F32)
    return _tn_x(r, ones11)


def _wsplit(w):
    hi = w.astype(_BF16)
    lo = (w - hi.astype(_F32)).astype(_BF16)
    return (hi, lo)


def _block_w(p):
    return {'ln1': p['ln1'].reshape(1, -1), 'ln2': p['ln2'].reshape(1, -1),
            'wq': _wsplit(p['wq']['w']), 'wk': _wsplit(p['wk']['w']),
            'wv': _wsplit(p['wv']['w']), 'wo': _wsplit(p['wo']['w']),
            'fc': _wsplit(p['fc']['w']), 'pr': _wsplit(p['pr']['w'])}


def _sblock_w(p):
    return {'ln1': p['ln1'].reshape(1, -1), 'ln2': p['ln2'].reshape(1, -1),
            'ln3': p['ln3'].reshape(1, -1),
            'q': _wsplit(p['q']['w']), 'k': _wsplit(p['k']['w']),
            'v': _wsplit(p['v']['w']), 'o': _wsplit(p['o']['w']),
            'fc': _wsplit(p['fc']['w']), 'pr': _wsplit(p['pr']['w'])}


_BLOCK_KEYS = ('ln1', 'ln2', 'wq', 'wk', 'wv', 'wo', 'fc', 'pr')
_SBLOCK_KEYS = ('ln1', 'ln2', 'ln3', 'q', 'k', 'v', 'o', 'fc', 'pr')


def _flat(vals):
    out = []
    for v in vals:
        if isinstance(v, tuple):
            out.extend(v)
        else:
            out.append(v)
    return out


def _wspec_for(vals):
    return [_wspec(v.shape) for v in _flat(vals)]


def _load(template, it):
    out = []
    for v in template:
        if isinstance(v, tuple):
            h = next(it)[...]
            l = next(it)[...]
            out.append((h, l))
        else:
            out.append(next(it)[...])
    return out


def _wspec(shape):
    nd = len(shape)
    return pl.BlockSpec(shape, lambda b: (0,) * nd)


def _bspec(shape3):
    return pl.BlockSpec((1,) + shape3[1:], lambda b: (b, 0, 0))


def _sc_embed(table, idx_flat):
    n = idx_flat.shape[0]
    d = table.shape[1]
    info = plsc.get_sparse_core_info()
    nc, ns = info.num_cores, info.num_subcores
    nw = nc * ns
    per_w = n // nw
    mesh = plsc.VectorSubcoreMesh(core_axis_name="c", subcore_axis_name="s")

    @functools.partial(
        pl.kernel, mesh=mesh,
        out_type=jax.ShapeDtypeStruct((n, d), _F32),
        scratch_types=[
            pltpu.VMEM((per_w,), jnp.int32),
            pltpu.VMEM((per_w, d), _F32),
            pltpu.SemaphoreType.DMA,
        ],
    )
    def emb_k(table_hbm, idx_hbm, out_hbm, idx_v, rows_v, sem):
        wid = lax.axis_index("s") * nc + lax.axis_index("c")
        base = wid * per_w
        pltpu.sync_copy(idx_hbm.at[pl.ds(base, per_w)], idx_v)
        pltpu.async_copy(table_hbm.at[idx_v], rows_v, sem).wait()
        pltpu.sync_copy(rows_v, out_hbm.at[pl.ds(base, per_w)])

    return emb_k(table, idx_flat)


def _k_up(x, pos, bw, upw, upb, iseq, add_pos, c_used=None):
    bsz, t, c = x.shape
    if c_used is None:
        c_used = c
    idim = upw[0].shape[1]
    wvals = [bw[kk] for kk in _BLOCK_KEYS] + [upw, upb]

    def body(x_ref, pos_ref, *rest):
        it = iter(rest[:-2])
        loaded = _load(wvals, it)
        scan_ref, pup_ref = rest[-2], rest[-1]
        w = dict(zip(_BLOCK_KEYS, loaded[:-2]))
        upw_v, upb_v = loaded[-2], loaded[-1]
        xx = x_ref[0][:, 0:c_used]
        if add_pos:
            xx = xx + pos_ref[...]
        scan = _block(w, xx)
        scan_ref[0] = scan
        g = _gate_cols(xx, scan)
        q = _sel_matrix(g, iseq)
        gathered = _tn_x(q, _softmax(scan))
        pup_ref[0] = _mm(gathered, upw_v) + upb_v

    in_specs = ([_bspec(x.shape), _wspec(pos.shape)]
                + _wspec_for(wvals))
    return pl.pallas_call(
        body,
        grid=(bsz,),
        in_specs=in_specs,
        out_specs=[_bspec((bsz, t, c_used)), _bspec((bsz, iseq, idim))],
        out_shape=[jax.ShapeDtypeStruct((bsz, t, c_used), _F32),
                   jax.ShapeDtypeStruct((bsz, iseq, idim), _F32)],
    )(x, pos, *_flat(wvals))


def _k_attn_flat(x, bw, blk):
    n, c = x.shape
    hd = c // NHEAD
    wvals = [bw['ln1'], bw['wq'], bw['wk'], bw['wv'], bw['wo']]

    def body(x_ref, *rest):
        loaded = _load(wvals, iter(rest[:-1]))
        out_ref = rest[-1]
        ln1, wq, wk, wv, wo = loaded
        x_v = x_ref[...]
        xn = _ln(ln1, x_v)
        q = _mm(xn, wq)
        k = _mm(xn, wk)
        v = _mm(xn, wv)
        ii = lax.broadcasted_iota(jnp.int32, (n, n), 0)
        jj = lax.broadcasted_iota(jnp.int32, (n, n), 1)
        mask = (ii // blk == jj // blk) & (jj <= ii)
        acc = x_v
        for h in range(NHEAD):
            sl = slice(h * hd, (h + 1) * hd)
            att = _nt(q[:, sl], k[:, sl]) / math.sqrt(float(hd))
            att = jnp.where(mask, att, -1e30)
            yh = _mm(_softmax(att), v[:, sl])
            acc = acc + _mm(yh, _pair_rows(wo, h * hd, (h + 1) * hd))
        out_ref[...] = acc

    return pl.pallas_call(
        body,
        grid=(1,),
        in_specs=[_wspec(x.shape)] + _wspec_for(wvals),
        out_specs=_wspec((n, c)),
        out_shape=jax.ShapeDtypeStruct((n, c), _F32),
    )(x, *_flat(wvals))


def _k_xattn_flat(xkv, sx, sbw, qblk, kvblk):
    nk, c = xkv.shape
    nq = sx.shape[0]
    hd = c // NHEAD
    wvals = [sbw['ln1'], sbw['ln2'], sbw['q'], sbw['k'], sbw['v'], sbw['o']]

    def body(xkv_ref, sx_ref, *rest):
        loaded = _load(wvals, iter(rest[:-1]))
        out_ref = rest[-1]
        ln1, ln2, wq, wk, wv, wo = loaded
        sx_v = sx_ref[...]
        xn = _ln(ln1, xkv_ref[...])
        sn = _ln(ln2, sx_v)
        q = _mm(sn, wq)
        k = _mm(xn, wk)
        v = _mm(xn, wv)
        ii = lax.broadcasted_iota(jnp.int32, (nq, nk), 0)
        jj = lax.broadcasted_iota(jnp.int32, (nq, nk), 1)
        mask = (ii // qblk) == (jj // kvblk)
        acc = sx_v
        for h in range(NHEAD):
            sl = slice(h * hd, (h + 1) * hd)
            att = _nt(q[:, sl], k[:, sl])
            att = jnp.where(mask, att, -1e30)
            yh = _mm(_softmax(att), v[:, sl])
            acc = acc + _mm(yh, _pair_rows(wo, h * hd, (h + 1) * hd))
        out_ref[...] = acc

    return pl.pallas_call(
        body,
        grid=(1,),
        in_specs=[_wspec(xkv.shape), _wspec(sx.shape)] + _wspec_for(wvals),
        out_specs=_wspec((nq, c)),
        out_shape=jax.ShapeDtypeStruct((nq, c), _F32),
    )(xkv, sx, *_flat(wvals))


def _k_mlp_tiled(x, lnw, fc, pr, ntiles):
    n, c = x.shape
    hdim = fc[0].shape[1]
    ht = hdim // ntiles

    def body(x_ref, lnw_ref, fch_ref, fcl_ref, prh_ref, prl_ref, out_ref):
        t = pl.program_id(0)
        xn = _ln(lnw_ref[...], x_ref[...])
        hcur = jax.nn.gelu(_mm(xn, (fch_ref[...], fcl_ref[...])))
        term = _mm(hcur, (prh_ref[...], prl_ref[...]))

        @pl.when(t == 0)
        def _():
            out_ref[...] = x_ref[...] + term

        @pl.when(t != 0)
        def _():
            out_ref[...] = out_ref[...] + term

    return pl.pallas_call(
        body,
        grid=(ntiles,),
        in_specs=[
            _wspec(x.shape), _wspec(lnw.shape),
            pl.BlockSpec((c, ht), lambda t: (0, t)),
            pl.BlockSpec((c, ht), lambda t: (0, t)),
            pl.BlockSpec((ht, c), lambda t: (t, 0)),
            pl.BlockSpec((ht, c), lambda t: (t, 0)),
        ],
        out_specs=_wspec((n, c)),
        out_shape=jax.ShapeDtypeStruct((n, c), _F32),
    )(x, lnw, fc[0], fc[1], pr[0], pr[1])


def _k_scatter(passed, w3t, sgb, posw, posb, oseq):
    bsz, iseq, idim = passed.shape

    def body(p_ref, w3t_ref, sgb_ref, posw_ref, posb_ref, out_ref):
        p = p_ref[0]
        sg_row = _sgate_row(p, w3t_ref[...], sgb_ref[...])
        sg_col = _row_to_col(sg_row)
        q = _sel_matrix(sg_col, iseq)
        out_ref[0] = _mm_x(q, p) + posw_ref[...] + posb_ref[...]

    return pl.pallas_call(
        body,
        grid=(bsz,),
        in_specs=[_bspec(passed.shape), _wspec(w3t.shape), _wspec(sgb.shape),
                  _wspec(posw.shape), _wspec(posb.shape)],
        out_specs=_bspec((bsz, oseq, idim)),
        out_shape=jax.ShapeDtypeStruct((bsz, oseq, idim), _F32),
    )(passed, w3t, sgb, posw, posb)


def _k_proj_block(pds, scan, dpw, dpb, dsw):
    bsz, t, idim = pds.shape
    od = scan.shape[2]
    wvals = [dpw, dpb] + [dsw[kk] for kk in _BLOCK_KEYS]

    def body(pds_ref, scan_ref, *rest):
        loaded = _load(wvals, iter(rest[:-1]))
        out_ref = rest[-1]
        dpw_v, dpb_v = loaded[0], loaded[1]
        dw = dict(zip(_BLOCK_KEYS, loaded[2:]))
        x = _mm(pds_ref[0], dpw_v) + dpb_v + scan_ref[0]
        out_ref[0] = _block(dw, x)

    return pl.pallas_call(
        body,
        grid=(bsz,),
        in_specs=[_bspec(pds.shape), _bspec(scan.shape)] + _wspec_for(wvals),
        out_specs=_bspec((bsz, t, od)),
        out_shape=jax.ShapeDtypeStruct((bsz, t, od), _F32),
    )(pds, scan, *_flat(wvals))


def _k_down_last(passed, scan, w3t, sgb, posw, posb, sbw, dpw, dpb, dsw):
    bsz, iseq, idim = passed.shape
    _, oseq, od = scan.shape
    wvals = ([w3t, sgb, posw, posb]
             + [sbw[kk] for kk in _SBLOCK_KEYS]
             + [dpw, dpb]
             + [dsw[kk] for kk in _BLOCK_KEYS])

    def body(p_ref, scan_ref, *rest):
        loaded = _load(wvals, iter(rest[:-1]))
        out_ref = rest[-1]
        w3t_v, sgb_v, posw_v, posb_v = loaded[:4]
        sw = dict(zip(_SBLOCK_KEYS, loaded[4:4 + len(_SBLOCK_KEYS)]))
        dpw_v, dpb_v = loaded[4 + len(_SBLOCK_KEYS):6 + len(_SBLOCK_KEYS)]
        dw = dict(zip(_BLOCK_KEYS, loaded[6 + len(_SBLOCK_KEYS):]))
        p = p_ref[0]
        sg_row = _sgate_row(p, w3t_v, sgb_v)
        sg_col = _row_to_col(sg_row)
        q = _sel_matrix(sg_col, iseq)
        scattered = _mm_x(q, p) + posw_v + posb_v
        pds = _sblock(sw, p, scattered)
        x = _mm(pds, dpw_v) + dpb_v + scan_ref[0]
        xn = _ln(dw['ln1'], x)
        hd = od // NHEAD
        k = _mm(xn, dw['wk'])
        v = _mm(xn, dw['wv'])
        q_last = _mm(xn[oseq - 1:oseq], dw['wq'])
        ys = []
        for h in range(NHEAD):
            sl = slice(h * hd, (h + 1) * hd)
            att = _nt(q_last[:, sl], k[:, sl]) / math.sqrt(float(hd))
            ys.append(_mm(_softmax(att), v[:, sl]))
        y = jnp.concatenate(ys, axis=1)
        xl = x[oseq - 1:oseq] + _mm(y, dw['wo'])
        out_ref[0] = xl + _mlp(dw, _ln(dw['ln2'], xl))

    return pl.pallas_call(
        body,
        grid=(bsz,),
        in_specs=[_bspec(passed.shape), _bspec(scan.shape)]
        + _wspec_for(wvals),
        out_specs=_bspec((bsz, 1, od)),
        out_shape=jax.ShapeDtypeStruct((bsz, 1, od), _F32),
    )(passed, scan, *_flat(wvals))


def _k_head(x, hw, hb):
    bsz, c = x.shape
    vocab = hw[0].shape[1]

    def body(x_ref, hwh_ref, hwl_ref, hb_ref, out_ref):
        out_ref[...] = _mm(x_ref[...], (hwh_ref[...], hwl_ref[...])) \
            + hb_ref[...]

    return pl.pallas_call(
        body,
        grid=(1,),
        in_specs=[_wspec(x.shape), _wspec(hw[0].shape), _wspec(hw[1].shape),
                  _wspec(hb.shape)],
        out_specs=_wspec((bsz, vocab)),
        out_shape=jax.ShapeDtypeStruct((bsz, vocab), _F32),
    )(x, hw[0], hw[1], hb)


def kernel(params, idx):
    bsz, t = idx.shape
    l1, l2 = params['L1'], params['L2']
    dim1 = params['tok_emb'].shape[1]
    k1 = l2['pos']['w'].shape[0]
    k2 = l2['s_gate']['w'].shape[0] // SDIM
    dim3 = l2['up_proj']['w'].shape[1]
    dim2 = l1['up_proj']['w'].shape[1]

    emb_w = max(128, dim1)
    table = jnp.pad(params['tok_emb'], ((0, 0), (0, emb_w - dim1)))
    emb = _sc_embed(table, idx.reshape(-1).astype(jnp.int32))
    emb = emb.reshape(bsz, t, emb_w)

    scan1, p_up1 = _k_up(
        emb, params['pos_emb'], _block_w(l1['up_scan']),
        _wsplit(l1['up_proj']['w']), l1['up_proj']['b'].reshape(1, -1),
        iseq=k1, add_pos=True, c_used=dim1)

    dummy_pos = jnp.zeros((1, 1), _F32)
    scan2, p_up2 = _k_up(
        p_up1, dummy_pos, _block_w(l2['up_scan']),
        _wsplit(l2['up_proj']['w']), l2['up_proj']['b'].reshape(1, -1),
        iseq=k2, add_pos=False)

    ib = _block_w(params['innerb'])
    xf = p_up2.reshape(bsz * k2, dim3)
    xf = _k_attn_flat(xf, ib, blk=k2)
    xf = _k_mlp_tiled(xf, ib['ln2'], ib['fc'], ib['pr'], ntiles=4)
    passed2 = xf.reshape(bsz, k2, dim3)

    w3t2 = l2['s_gate']['w'].reshape(k2, SDIM, k1).transpose(1, 0, 2)
    scattered2 = _k_scatter(
        passed2, w3t2, l2['s_gate']['b'].reshape(1, -1),
        l2['pos']['w'], l2['pos']['b'].reshape(1, -1), oseq=k1)
    sb2 = _sblock_w(l2['down_scatter'])
    sx1 = _k_xattn_flat(xf, scattered2.reshape(bsz * k1, dim3), sb2,
                        qblk=k1, kvblk=k2)
    pds2 = _k_mlp_tiled(sx1, sb2['ln3'], sb2['fc'], sb2['pr'], ntiles=4)
    passed1 = _k_proj_block(
        pds2.reshape(bsz, k1, dim3), scan2,
        _wsplit(l2['down_proj']['w']), l2['down_proj']['b'].reshape(1, -1),
        _block_w(l2['down_scan']))

    w3t1 = l1['s_gate']['w'].reshape(k1, SDIM, t).transpose(1, 0, 2)
    out_last = _k_down_last(
        passed1, scan1, w3t1, l1['s_gate']['b'].reshape(1, -1),
        l1['pos']['w'], l1['pos']['b'].reshape(1, -1),
        _sblock_w(l1['down_scatter']),
        _wsplit(l1['down_proj']['w']), l1['down_proj']['b'].reshape(1, -1),
        _block_w(l1['down_scan']))

    logits = _k_head(out_last.reshape(bsz, dim1),
                     _wsplit(params['head']['w']),
                     params['head']['b'].reshape(1, -1))
    return logits.reshape(bsz, 1, logits.shape[1])
